# baseline (speedup 1.0000x reference)
import functools

import jax
import jax.numpy as jnp
from jax import lax
from jax.experimental import pallas as pl
from jax.experimental.pallas import tpu as pltpu
from jax.experimental.pallas import tpu_sc as plsc

_NW = 32
_LANES = 16


def _lpe_body(L, B, nblk_total, u_hbm, v_hbm, mu_hbm, mv_hbm, out_hbm,
              u_v, v_v, o_v, mu_v, mv_v, sem_in, sem_out, sem_tab):
    cid = lax.axis_index("c")
    sid = lax.axis_index("s")
    wid = sid * 2 + cid
    vpb = B // _LANES
    kmax = (nblk_total + _NW - 1) // _NW

    tab_cp = (pltpu.async_copy(mu_hbm, mu_v, sem_tab),
              pltpu.async_copy(mv_hbm, mv_v, sem_tab))

    iota = lax.iota(jnp.int32, _LANES)
    z16 = jnp.zeros((_LANES,), jnp.int32)
    maxi = jnp.full((_LANES,), L - 1, jnp.int32)
    cL = jnp.full((_LANES,), L, jnp.int32)
    fscale = jnp.float32(L - 1)

    def in_copies(b, buf):
        sl = pl.ds(b * B, B)
        return (pltpu.async_copy(u_hbm.at[sl], u_v.at[buf], sem_in[buf]),
                pltpu.async_copy(v_hbm.at[sl], v_v.at[buf], sem_in[buf]))

    def lerp2(tab_v, i0, i1, w):
        a0 = plsc.load_gather(tab_v, [i0])
        a1 = plsc.load_gather(tab_v, [i1])
        b0 = plsc.load_gather(tab_v, [i0 + cL])
        b1 = plsc.load_gather(tab_v, [i1 + cL])
        return a0 + w * (a1 - a0), b0 + w * (b1 - b0)

    def make_vec_body(buf):
        def vec_body(jj):
            sl = pl.ds(jj * _LANES, _LANES)
            u = u_v[buf, sl]
            v = v_v[buf, sl]

            tu = (u + 1.0) * 0.5 * fscale
            tv = (v + 1.0) * 0.5 * fscale
            iu0 = tu.astype(jnp.int32)
            iv0 = tv.astype(jnp.int32)
            wu = tu - iu0.astype(jnp.float32)
            wv = tv - iv0.astype(jnp.float32)
            iu0 = jnp.minimum(jnp.maximum(iu0, z16), maxi)
            iv0 = jnp.minimum(jnp.maximum(iv0, z16), maxi)
            iu1 = jnp.minimum(iu0 + 1, maxi)
            iv1 = jnp.minimum(iv0 + 1, maxi)

            mu0, mu1 = lerp2(mu_v, iu0, iu1, wu)
            mv0, mv1 = lerp2(mv_v, iv0, iv1, wv)

            p4 = (jj * _LANES + iota) * 4
            ov = o_v.at[buf]
            plsc.store_scatter(ov, [p4], mu0)
            plsc.store_scatter(ov, [p4 + 1], mu1)
            plsc.store_scatter(ov, [p4 + 2], mv0)
            plsc.store_scatter(ov, [p4 + 3], mv1)
        return vec_body

    @pl.when(wid < nblk_total)
    def _():
        in_copies(wid, 0)

    tab_cp[0].wait()
    tab_cp[1].wait()

    for k in range(kmax):
        cur = k % 2
        b = k * _NW + wid

        if k + 1 < kmax:
            bn = (k + 1) * _NW + wid

            @pl.when(bn < nblk_total)
            def _(bn=bn, nxt=1 - cur):
                in_copies(bn, nxt)

        @pl.when(b < nblk_total)
        def _(k=k, b=b, cur=cur):
            sl = pl.ds(b * B, B)
            pltpu.make_async_copy(u_hbm.at[sl], u_v.at[cur], sem_in[cur]).wait()
            pltpu.make_async_copy(v_hbm.at[sl], v_v.at[cur], sem_in[cur]).wait()
            if k >= 2:
                bp = (k - 2) * _NW + wid
                slp = pl.ds(bp * B * 4, B * 4)
                pltpu.make_async_copy(
                    o_v.at[cur], out_hbm.at[slp], sem_out[cur]).wait()
            plsc.parallel_loop(0, vpb, unroll=8)(make_vec_body(cur))
            pltpu.async_copy(o_v.at[cur], out_hbm.at[pl.ds(b * B * 4, B * 4)],
                             sem_out[cur])

    for k in (kmax - 2, kmax - 1):
        if k >= 0:
            b = k * _NW + wid

            @pl.when(b < nblk_total)
            def _(k=k, b=b):
                pltpu.make_async_copy(
                    o_v.at[k % 2], out_hbm.at[pl.ds(b * B * 4, B * 4)],
                    sem_out[k % 2]).wait()


def kernel(uv, m_u, m_v):
    N = uv.shape[0]
    L = m_u.shape[1]
    B = 4000
    nblk_total = N // B

    mesh = plsc.VectorSubcoreMesh(core_axis_name="c", subcore_axis_name="s")
    f = pl.kernel(
        functools.partial(_lpe_body, L, B, nblk_total),
        out_type=jax.ShapeDtypeStruct((N * 4,), jnp.float32),
        mesh=mesh,
        compiler_params=pltpu.CompilerParams(
            needs_layout_passes=False, use_tc_tiling_on_sc=False),
        scratch_types=[
            pltpu.VMEM((2, B), jnp.float32),
            pltpu.VMEM((2, B), jnp.float32),
            pltpu.VMEM((2, 4 * B), jnp.float32),
            pltpu.VMEM((2 * L,), jnp.float32),
            pltpu.VMEM((2 * L,), jnp.float32),
            (pltpu.SemaphoreType.DMA, pltpu.SemaphoreType.DMA),
            (pltpu.SemaphoreType.DMA, pltpu.SemaphoreType.DMA),
            pltpu.SemaphoreType.DMA,
        ],
    )
    out_flat = f(uv[:, 0], uv[:, 1],
                 m_u.reshape(2 * L), m_v.reshape(2 * L))
    return out_flat.reshape(N, 4)


# speedup vs baseline: 3.9151x; 3.9151x over previous
import functools

import jax
import jax.numpy as jnp
from jax import lax
from jax.experimental import pallas as pl
from jax.experimental.pallas import tpu as pltpu
from jax.experimental.pallas import tpu_sc as plsc

_NW = 32
_LANES = 16


def _lpe_body(L, B, nblk_total, u_hbm, v_hbm, mu_hbm, mv_hbm,
              o0_hbm, o1_hbm, o2_hbm, o3_hbm,
              u_v, v_v, o_v, mu_v, mv_v, sem_in, sem_out, sem_tab):
    cid = lax.axis_index("c")
    sid = lax.axis_index("s")
    wid = sid * 2 + cid
    vpb = B // _LANES
    kmax = (nblk_total + _NW - 1) // _NW

    tab_cp = (pltpu.async_copy(mu_hbm, mu_v, sem_tab),
              pltpu.async_copy(mv_hbm, mv_v, sem_tab))

    z16 = jnp.zeros((_LANES,), jnp.int32)
    maxi = jnp.full((_LANES,), L - 1, jnp.int32)
    cL = jnp.full((_LANES,), L, jnp.int32)
    fscale = jnp.float32(L - 1)

    def in_copies(b, buf):
        sl = pl.ds(b * B, B)
        return (pltpu.async_copy(u_hbm.at[sl], u_v.at[buf], sem_in[buf]),
                pltpu.async_copy(v_hbm.at[sl], v_v.at[buf], sem_in[buf]))

    def out_copies(b, buf):
        sl = pl.ds(b * B, B)
        return (pltpu.async_copy(o_v.at[buf, 0], o0_hbm.at[sl], sem_out[buf]),
                pltpu.async_copy(o_v.at[buf, 1], o1_hbm.at[sl], sem_out[buf]),
                pltpu.async_copy(o_v.at[buf, 2], o2_hbm.at[sl], sem_out[buf]),
                pltpu.async_copy(o_v.at[buf, 3], o3_hbm.at[sl], sem_out[buf]))

    def lerp2(tab_v, i0, i1, w):
        a0 = plsc.load_gather(tab_v, [i0])
        a1 = plsc.load_gather(tab_v, [i1])
        b0 = plsc.load_gather(tab_v, [i0 + cL])
        b1 = plsc.load_gather(tab_v, [i1 + cL])
        return a0 + w * (a1 - a0), b0 + w * (b1 - b0)

    def make_vec_body(buf):
        def vec_body(jj):
            sl = pl.ds(jj * _LANES, _LANES)
            u = u_v[buf, sl]
            v = v_v[buf, sl]

            tu = (u + 1.0) * 0.5 * fscale
            tv = (v + 1.0) * 0.5 * fscale
            iu0 = tu.astype(jnp.int32)
            iv0 = tv.astype(jnp.int32)
            wu = tu - iu0.astype(jnp.float32)
            wv = tv - iv0.astype(jnp.float32)
            iu0 = jnp.minimum(jnp.maximum(iu0, z16), maxi)
            iv0 = jnp.minimum(jnp.maximum(iv0, z16), maxi)
            iu1 = jnp.minimum(iu0 + 1, maxi)
            iv1 = jnp.minimum(iv0 + 1, maxi)

            mu0, mu1 = lerp2(mu_v, iu0, iu1, wu)
            mv0, mv1 = lerp2(mv_v, iv0, iv1, wv)

            o_v[buf, 0, sl] = mu0
            o_v[buf, 1, sl] = mu1
            o_v[buf, 2, sl] = mv0
            o_v[buf, 3, sl] = mv1
        return vec_body

    @pl.when(wid < nblk_total)
    def _():
        in_copies(wid, 0)

    tab_cp[0].wait()
    tab_cp[1].wait()

    for k in range(kmax):
        cur = k % 2
        b = k * _NW + wid

        if k + 1 < kmax:
            bn = (k + 1) * _NW + wid

            @pl.when(bn < nblk_total)
            def _(bn=bn, nxt=1 - cur):
                in_copies(bn, nxt)

        @pl.when(b < nblk_total)
        def _(k=k, b=b, cur=cur):
            sl = pl.ds(b * B, B)
            pltpu.make_async_copy(u_hbm.at[sl], u_v.at[cur], sem_in[cur]).wait()
            pltpu.make_async_copy(v_hbm.at[sl], v_v.at[cur], sem_in[cur]).wait()
            if k >= 2:
                bp = (k - 2) * _NW + wid
                slp = pl.ds(bp * B, B)
                for i, oh in enumerate((o0_hbm, o1_hbm, o2_hbm, o3_hbm)):
                    pltpu.make_async_copy(
                        o_v.at[cur, i], oh.at[slp], sem_out[cur]).wait()
            plsc.parallel_loop(0, vpb, unroll=8)(make_vec_body(cur))
            out_copies(b, cur)

    for k in (kmax - 2, kmax - 1):
        if k >= 0:
            b = k * _NW + wid

            @pl.when(b < nblk_total)
            def _(k=k, b=b):
                sl = pl.ds(b * B, B)
                for i, oh in enumerate((o0_hbm, o1_hbm, o2_hbm, o3_hbm)):
                    pltpu.make_async_copy(
                        o_v.at[k % 2, i], oh.at[sl], sem_out[k % 2]).wait()


def kernel(uv, m_u, m_v):
    N = uv.shape[0]
    L = m_u.shape[1]
    B = 4000
    nblk_total = N // B

    mesh = plsc.VectorSubcoreMesh(core_axis_name="c", subcore_axis_name="s")
    plane = jax.ShapeDtypeStruct((N,), jnp.float32)
    f = pl.kernel(
        functools.partial(_lpe_body, L, B, nblk_total),
        out_type=(plane, plane, plane, plane),
        mesh=mesh,
        compiler_params=pltpu.CompilerParams(
            needs_layout_passes=False, use_tc_tiling_on_sc=False),
        scratch_types=[
            pltpu.VMEM((2, B), jnp.float32),
            pltpu.VMEM((2, B), jnp.float32),
            pltpu.VMEM((2, 4, B), jnp.float32),
            pltpu.VMEM((2 * L,), jnp.float32),
            pltpu.VMEM((2 * L,), jnp.float32),
            (pltpu.SemaphoreType.DMA, pltpu.SemaphoreType.DMA),
            (pltpu.SemaphoreType.DMA, pltpu.SemaphoreType.DMA),
            pltpu.SemaphoreType.DMA,
        ],
    )
    o0, o1, o2, o3 = f(uv[:, 0], uv[:, 1],
                       m_u.reshape(2 * L), m_v.reshape(2 * L))
    return jnp.stack([o0, o1, o2, o3], axis=1)


# speedup vs baseline: 6.0444x; 1.5439x over previous
import functools

import jax
import jax.numpy as jnp
from jax import lax
from jax.experimental import pallas as pl
from jax.experimental.pallas import tpu as pltpu
from jax.experimental.pallas import tpu_sc as plsc

_NW = 32
_LANES = 16
_G = 128


def _lpe_body(L, B, nblk_total, u_hbm, v_hbm, mu_hbm, mv_hbm, out_hbm,
              u_v, v_v, o_v, mu_v, mv_v, sem_in, sem_out, sem_tab):
    cid = lax.axis_index("c")
    sid = lax.axis_index("s")
    wid = sid * 2 + cid
    vpb = B // _LANES
    kmax = (nblk_total + _NW - 1) // _NW

    tab_cp = (pltpu.async_copy(mu_hbm, mu_v, sem_tab),
              pltpu.async_copy(mv_hbm, mv_v, sem_tab))

    z16 = jnp.zeros((_LANES,), jnp.int32)
    maxi = jnp.full((_LANES,), L - 1, jnp.int32)
    cL = jnp.full((_LANES,), L, jnp.int32)
    fscale = jnp.float32(L - 1)

    def in_copies(b, buf):
        sl = pl.ds(b * B, B)
        return (pltpu.async_copy(u_hbm.at[sl], u_v.at[buf], sem_in[buf]),
                pltpu.async_copy(v_hbm.at[sl], v_v.at[buf], sem_in[buf]))

    def lerp2(tab_v, i0, i1, w):
        a0 = plsc.load_gather(tab_v, [i0])
        a1 = plsc.load_gather(tab_v, [i1])
        b0 = plsc.load_gather(tab_v, [i0 + cL])
        b1 = plsc.load_gather(tab_v, [i1 + cL])
        return a0 + w * (a1 - a0), b0 + w * (b1 - b0)

    def make_vec_body(buf):
        def vec_body(jj):
            sl = pl.ds(jj * _LANES, _LANES)
            u = u_v[buf, sl]
            v = v_v[buf, sl]

            tu = (u + 1.0) * 0.5 * fscale
            tv = (v + 1.0) * 0.5 * fscale
            iu0 = tu.astype(jnp.int32)
            iv0 = tv.astype(jnp.int32)
            wu = tu - iu0.astype(jnp.float32)
            wv = tv - iv0.astype(jnp.float32)
            iu0 = jnp.minimum(jnp.maximum(iu0, z16), maxi)
            iv0 = jnp.minimum(jnp.maximum(iv0, z16), maxi)
            iu1 = jnp.minimum(iu0 + 1, maxi)
            iv1 = jnp.minimum(iv0 + 1, maxi)

            mu0, mu1 = lerp2(mu_v, iu0, iu1, wu)
            mv0, mv1 = lerp2(mv_v, iv0, iv1, wv)

            base = (jj // 8) * (4 * _G) + (jj % 8) * _LANES
            o_v[buf, pl.ds(base, _LANES)] = mu0
            o_v[buf, pl.ds(base + _G, _LANES)] = mu1
            o_v[buf, pl.ds(base + 2 * _G, _LANES)] = mv0
            o_v[buf, pl.ds(base + 3 * _G, _LANES)] = mv1
        return vec_body

    @pl.when(wid < nblk_total)
    def _():
        in_copies(wid, 0)

    tab_cp[0].wait()
    tab_cp[1].wait()

    for k in range(kmax):
        cur = k % 2
        b = k * _NW + wid

        if k + 1 < kmax:
            bn = (k + 1) * _NW + wid

            @pl.when(bn < nblk_total)
            def _(bn=bn, nxt=1 - cur):
                in_copies(bn, nxt)

        @pl.when(b < nblk_total)
        def _(k=k, b=b, cur=cur):
            sl = pl.ds(b * B, B)
            pltpu.make_async_copy(u_hbm.at[sl], u_v.at[cur], sem_in[cur]).wait()
            pltpu.make_async_copy(v_hbm.at[sl], v_v.at[cur], sem_in[cur]).wait()
            if k >= 2:
                bp = (k - 2) * _NW + wid
                slp = pl.ds(bp * B * 4, B * 4)
                pltpu.make_async_copy(
                    o_v.at[cur], out_hbm.at[slp], sem_out[cur]).wait()
            plsc.parallel_loop(0, vpb, unroll=8)(make_vec_body(cur))
            pltpu.async_copy(o_v.at[cur], out_hbm.at[pl.ds(b * B * 4, B * 4)],
                             sem_out[cur])

    for k in (kmax - 2, kmax - 1):
        if k >= 0:
            b = k * _NW + wid

            @pl.when(b < nblk_total)
            def _(k=k, b=b):
                pltpu.make_async_copy(
                    o_v.at[k % 2], out_hbm.at[pl.ds(b * B * 4, B * 4)],
                    sem_out[k % 2]).wait()


def kernel(uv, m_u, m_v):
    N = uv.shape[0]
    L = m_u.shape[1]
    G = _G
    B = 4480
    Np = ((N + B * _NW - 1) // (B * _NW)) * (B * _NW)
    nblk_total = Np // B

    pad = Np - N
    u = jnp.pad(uv[:, 0], (0, pad))
    v = jnp.pad(uv[:, 1], (0, pad))

    mesh = plsc.VectorSubcoreMesh(core_axis_name="c", subcore_axis_name="s")
    f = pl.kernel(
        functools.partial(_lpe_body, L, B, nblk_total),
        out_type=jax.ShapeDtypeStruct((Np * 4,), jnp.float32),
        mesh=mesh,
        compiler_params=pltpu.CompilerParams(
            needs_layout_passes=False, use_tc_tiling_on_sc=False),
        scratch_types=[
            pltpu.VMEM((2, B), jnp.float32),
            pltpu.VMEM((2, B), jnp.float32),
            pltpu.VMEM((2, 4 * B), jnp.float32),
            pltpu.VMEM((2 * L,), jnp.float32),
            pltpu.VMEM((2 * L,), jnp.float32),
            (pltpu.SemaphoreType.DMA, pltpu.SemaphoreType.DMA),
            (pltpu.SemaphoreType.DMA, pltpu.SemaphoreType.DMA),
            pltpu.SemaphoreType.DMA,
        ],
    )
    out_flat = f(u, v, m_u.reshape(2 * L), m_v.reshape(2 * L))
    out = out_flat.reshape(Np // G, 4, G).transpose(0, 2, 1).reshape(Np, 4)
    return out[:N]


# speedup vs baseline: 9.2191x; 1.5252x over previous
import functools

import jax
import jax.numpy as jnp
from jax import lax
from jax.experimental import pallas as pl
from jax.experimental.pallas import tpu as pltpu
from jax.experimental.pallas import tpu_sc as plsc

_NW = 32
_LANES = 16
_G = 128


def _lpe_body(L, B, nblk_total, uv_hbm, mu_hbm, mv_hbm, out_hbm,
              uv_v, o_v, mu_v, mv_v, sem_in, sem_out, sem_tab):
    cid = lax.axis_index("c")
    sid = lax.axis_index("s")
    wid = sid * 2 + cid
    vpb = B // _LANES
    kmax = (nblk_total + _NW - 1) // _NW

    tab_cp = (pltpu.async_copy(mu_hbm, mu_v, sem_tab),
              pltpu.async_copy(mv_hbm, mv_v, sem_tab))

    z16 = jnp.zeros((_LANES,), jnp.int32)
    maxi = jnp.full((_LANES,), L - 1, jnp.int32)
    cL = jnp.full((_LANES,), L, jnp.int32)
    fscale = jnp.float32(L - 1)

    def in_copy(b, buf):
        sl = pl.ds(b * 2 * B, 2 * B)
        return pltpu.async_copy(uv_hbm.at[sl], uv_v.at[buf], sem_in[buf])

    def lerp2(tab_v, i0, i1, w):
        a0 = plsc.load_gather(tab_v, [i0])
        a1 = plsc.load_gather(tab_v, [i1])
        b0 = plsc.load_gather(tab_v, [i0 + cL])
        b1 = plsc.load_gather(tab_v, [i1 + cL])
        return a0 + w * (a1 - a0), b0 + w * (b1 - b0)

    def make_vec_body(buf):
        def vec_body(jj):
            ibase = (jj // 8) * (2 * _G) + (jj % 8) * _LANES
            u = uv_v[buf, pl.ds(ibase, _LANES)]
            v = uv_v[buf, pl.ds(ibase + _G, _LANES)]

            tu = (u + 1.0) * 0.5 * fscale
            tv = (v + 1.0) * 0.5 * fscale
            iu0 = tu.astype(jnp.int32)
            iv0 = tv.astype(jnp.int32)
            wu = tu - iu0.astype(jnp.float32)
            wv = tv - iv0.astype(jnp.float32)
            iu0 = jnp.minimum(jnp.maximum(iu0, z16), maxi)
            iv0 = jnp.minimum(jnp.maximum(iv0, z16), maxi)
            iu1 = jnp.minimum(iu0 + 1, maxi)
            iv1 = jnp.minimum(iv0 + 1, maxi)

            mu0, mu1 = lerp2(mu_v, iu0, iu1, wu)
            mv0, mv1 = lerp2(mv_v, iv0, iv1, wv)

            base = (jj // 8) * (4 * _G) + (jj % 8) * _LANES
            o_v[buf, pl.ds(base, _LANES)] = mu0
            o_v[buf, pl.ds(base + _G, _LANES)] = mu1
            o_v[buf, pl.ds(base + 2 * _G, _LANES)] = mv0
            o_v[buf, pl.ds(base + 3 * _G, _LANES)] = mv1
        return vec_body

    @pl.when(wid < nblk_total)
    def _():
        in_copy(wid, 0)

    tab_cp[0].wait()
    tab_cp[1].wait()

    for k in range(kmax):
        cur = k % 2
        b = k * _NW + wid

        if k + 1 < kmax:
            bn = (k + 1) * _NW + wid

            @pl.when(bn < nblk_total)
            def _(bn=bn, nxt=1 - cur):
                in_copy(bn, nxt)

        @pl.when(b < nblk_total)
        def _(k=k, b=b, cur=cur):
            pltpu.make_async_copy(uv_hbm.at[pl.ds(b * 2 * B, 2 * B)],
                                  uv_v.at[cur], sem_in[cur]).wait()
            if k >= 2:
                bp = (k - 2) * _NW + wid
                slp = pl.ds(bp * B * 4, B * 4)
                pltpu.make_async_copy(
                    o_v.at[cur], out_hbm.at[slp], sem_out[cur]).wait()
            plsc.parallel_loop(0, vpb, unroll=8)(make_vec_body(cur))
            pltpu.async_copy(o_v.at[cur], out_hbm.at[pl.ds(b * B * 4, B * 4)],
                             sem_out[cur])

    for k in (kmax - 2, kmax - 1):
        if k >= 0:
            b = k * _NW + wid

            @pl.when(b < nblk_total)
            def _(k=k, b=b):
                pltpu.make_async_copy(
                    o_v.at[k % 2], out_hbm.at[pl.ds(b * B * 4, B * 4)],
                    sem_out[k % 2]).wait()


def kernel(uv, m_u, m_v):
    N = uv.shape[0]
    L = m_u.shape[1]
    G = _G
    B = 4480
    Np = ((N + B * _NW - 1) // (B * _NW)) * (B * _NW)
    nblk_total = Np // B

    pad = Np - N
    uvp = jnp.pad(uv, ((0, pad), (0, 0)))
    uvx = uvp.reshape(Np // G, G, 2).transpose(0, 2, 1).reshape(2 * Np)

    mesh = plsc.VectorSubcoreMesh(core_axis_name="c", subcore_axis_name="s")
    f = pl.kernel(
        functools.partial(_lpe_body, L, B, nblk_total),
        out_type=jax.ShapeDtypeStruct((Np * 4,), jnp.float32),
        mesh=mesh,
        compiler_params=pltpu.CompilerParams(
            needs_layout_passes=False, use_tc_tiling_on_sc=False),
        scratch_types=[
            pltpu.VMEM((2, 2 * B), jnp.float32),
            pltpu.VMEM((2, 4 * B), jnp.float32),
            pltpu.VMEM((2 * L,), jnp.float32),
            pltpu.VMEM((2 * L,), jnp.float32),
            (pltpu.SemaphoreType.DMA, pltpu.SemaphoreType.DMA),
            (pltpu.SemaphoreType.DMA, pltpu.SemaphoreType.DMA),
            pltpu.SemaphoreType.DMA,
        ],
    )
    out_flat = f(uvx, m_u.reshape(2 * L), m_v.reshape(2 * L))
    out = out_flat.reshape(Np // G, 4, G).transpose(0, 2, 1).reshape(Np, 4)
    return out[:N]


# speedup vs baseline: 9.8305x; 1.0663x over previous
import functools

import jax
import jax.numpy as jnp
from jax import lax
from jax.experimental import pallas as pl
from jax.experimental.pallas import tpu as pltpu
from jax.experimental.pallas import tpu_sc as plsc

_NW = 32
_LANES = 16
_G = 128


def _lpe_body(L, B, nblk_total, uv_hbm, mu_hbm, mv_hbm, out_hbm,
              uv_v, o_v, mu_v, mv_v, sem_in, sem_out, sem_tab):
    cid = lax.axis_index("c")
    sid = lax.axis_index("s")
    wid = sid * 2 + cid
    kmax = (nblk_total + _NW - 1) // _NW

    tab_cp = (pltpu.async_copy(mu_hbm, mu_v, sem_tab),
              pltpu.async_copy(mv_hbm, mv_v, sem_tab))

    z16 = jnp.zeros((_LANES,), jnp.int32)
    maxi = jnp.full((_LANES,), L - 1, jnp.int32)
    cL = jnp.full((_LANES,), L, jnp.int32)
    fscale = jnp.float32(L - 1)

    def in_copy(b, buf):
        sl = pl.ds(b * 2 * B, 2 * B)
        return pltpu.async_copy(uv_hbm.at[sl], uv_v.at[buf], sem_in[buf])

    def lerp2(tab_v, i0, i1, w):
        a0 = plsc.load_gather(tab_v, [i0])
        a1 = plsc.load_gather(tab_v, [i1])
        b0 = plsc.load_gather(tab_v, [i0 + cL])
        b1 = plsc.load_gather(tab_v, [i1 + cL])
        return a0 + w * (a1 - a0), b0 + w * (b1 - b0)

    def make_grp_body(buf):
        def grp_body(g):
            for i in range(_G // _LANES):
                q0 = i * _LANES
                u = uv_v[buf, pl.ds(g * 2 * _G + q0, _LANES)]
                v = uv_v[buf, pl.ds(g * 2 * _G + _G + q0, _LANES)]

                tu = (u + 1.0) * 0.5 * fscale
                tv = (v + 1.0) * 0.5 * fscale
                iu0 = tu.astype(jnp.int32)
                iv0 = tv.astype(jnp.int32)
                wu = tu - iu0.astype(jnp.float32)
                wv = tv - iv0.astype(jnp.float32)
                iu0 = jnp.minimum(jnp.maximum(iu0, z16), maxi)
                iv0 = jnp.minimum(jnp.maximum(iv0, z16), maxi)
                iu1 = jnp.minimum(iu0 + 1, maxi)
                iv1 = jnp.minimum(iv0 + 1, maxi)

                mu0, mu1 = lerp2(mu_v, iu0, iu1, wu)
                mv0, mv1 = lerp2(mv_v, iv0, iv1, wv)

                base = g * 4 * _G + q0
                o_v[buf, pl.ds(base, _LANES)] = mu0
                o_v[buf, pl.ds(base + _G, _LANES)] = mu1
                o_v[buf, pl.ds(base + 2 * _G, _LANES)] = mv0
                o_v[buf, pl.ds(base + 3 * _G, _LANES)] = mv1
        return grp_body

    @pl.when(wid < nblk_total)
    def _():
        in_copy(wid, 0)

    tab_cp[0].wait()
    tab_cp[1].wait()

    for k in range(kmax):
        cur = k % 2
        b = k * _NW + wid

        if k + 1 < kmax:
            bn = (k + 1) * _NW + wid

            @pl.when(bn < nblk_total)
            def _(bn=bn, nxt=1 - cur):
                in_copy(bn, nxt)

        @pl.when(b < nblk_total)
        def _(k=k, b=b, cur=cur):
            pltpu.make_async_copy(uv_hbm.at[pl.ds(b * 2 * B, 2 * B)],
                                  uv_v.at[cur], sem_in[cur]).wait()
            if k >= 2:
                bp = (k - 2) * _NW + wid
                slp = pl.ds(bp * B * 4, B * 4)
                pltpu.make_async_copy(
                    o_v.at[cur], out_hbm.at[slp], sem_out[cur]).wait()
            plsc.parallel_loop(0, B // _G, unroll=5)(make_grp_body(cur))
            pltpu.async_copy(o_v.at[cur], out_hbm.at[pl.ds(b * B * 4, B * 4)],
                             sem_out[cur])

    for k in (kmax - 2, kmax - 1):
        if k >= 0:
            b = k * _NW + wid

            @pl.when(b < nblk_total)
            def _(k=k, b=b):
                pltpu.make_async_copy(
                    o_v.at[k % 2], out_hbm.at[pl.ds(b * B * 4, B * 4)],
                    sem_out[k % 2]).wait()


def kernel(uv, m_u, m_v):
    N = uv.shape[0]
    L = m_u.shape[1]
    G = _G
    B = 4480
    Np = ((N + B * _NW - 1) // (B * _NW)) * (B * _NW)
    nblk_total = Np // B

    pad = Np - N
    uvp = jnp.pad(uv, ((0, pad), (0, 0)))
    uvx = uvp.reshape(Np // G, G, 2).transpose(0, 2, 1).reshape(2 * Np)

    mesh = plsc.VectorSubcoreMesh(core_axis_name="c", subcore_axis_name="s")
    f = pl.kernel(
        functools.partial(_lpe_body, L, B, nblk_total),
        out_type=jax.ShapeDtypeStruct((Np * 4,), jnp.float32),
        mesh=mesh,
        compiler_params=pltpu.CompilerParams(
            needs_layout_passes=False, use_tc_tiling_on_sc=False),
        scratch_types=[
            pltpu.VMEM((2, 2 * B), jnp.float32),
            pltpu.VMEM((2, 4 * B), jnp.float32),
            pltpu.VMEM((2 * L,), jnp.float32),
            pltpu.VMEM((2 * L,), jnp.float32),
            (pltpu.SemaphoreType.DMA, pltpu.SemaphoreType.DMA),
            (pltpu.SemaphoreType.DMA, pltpu.SemaphoreType.DMA),
            pltpu.SemaphoreType.DMA,
        ],
    )
    out_flat = f(uvx, m_u.reshape(2 * L), m_v.reshape(2 * L))
    out = out_flat.reshape(Np // G, 4, G).transpose(0, 2, 1).reshape(Np, 4)
    return out[:N]


# speedup vs baseline: 10.5136x; 1.0695x over previous
import functools

import jax
import jax.numpy as jnp
from jax import lax
from jax.experimental import pallas as pl
from jax.experimental.pallas import tpu as pltpu
from jax.experimental.pallas import tpu_sc as plsc

_NW = 32
_LANES = 16
_G = 128


def _lpe_body(L, B, nblk_total, uv_hbm, mu_hbm, mv_hbm, out_hbm,
              uv_v, o_v, mu_v, mv_v, sem_in, sem_out, sem_tab):
    cid = lax.axis_index("c")
    sid = lax.axis_index("s")
    wid = sid * 2 + cid
    kmax = (nblk_total + _NW - 1) // _NW

    tab_cp = (pltpu.async_copy(mu_hbm, mu_v, sem_tab),
              pltpu.async_copy(mv_hbm, mv_v, sem_tab))

    maxi = jnp.full((_LANES,), L - 1, jnp.int32)
    cL = jnp.full((_LANES,), L, jnp.int32)
    fscale = jnp.float32(L - 1)

    def in_copy(b, buf):
        sl = pl.ds(b * 2 * B, 2 * B)
        return pltpu.async_copy(uv_hbm.at[sl], uv_v.at[buf], sem_in[buf])

    def lerp2(tab_v, i0, i1, w):
        a0 = plsc.load_gather(tab_v, [i0])
        a1 = plsc.load_gather(tab_v, [i1])
        b0 = plsc.load_gather(tab_v, [i0 + cL])
        b1 = plsc.load_gather(tab_v, [i1 + cL])
        return a0 + w * (a1 - a0), b0 + w * (b1 - b0)

    def make_grp_body(buf):
        def grp_body(g):
            for i in range(_G // _LANES):
                q0 = i * _LANES
                u = uv_v[buf, pl.ds(g * 2 * _G + q0, _LANES)]
                v = uv_v[buf, pl.ds(g * 2 * _G + _G + q0, _LANES)]

                tu = (u + 1.0) * 0.5 * fscale
                tv = (v + 1.0) * 0.5 * fscale
                iu0 = tu.astype(jnp.int32)
                iv0 = tv.astype(jnp.int32)
                wu = tu - iu0.astype(jnp.float32)
                wv = tv - iv0.astype(jnp.float32)
                iu1 = jnp.minimum(iu0 + 1, maxi)
                iv1 = jnp.minimum(iv0 + 1, maxi)

                mu0, mu1 = lerp2(mu_v, iu0, iu1, wu)
                mv0, mv1 = lerp2(mv_v, iv0, iv1, wv)

                base = g * 4 * _G + q0
                o_v[buf, pl.ds(base, _LANES)] = mu0
                o_v[buf, pl.ds(base + _G, _LANES)] = mu1
                o_v[buf, pl.ds(base + 2 * _G, _LANES)] = mv0
                o_v[buf, pl.ds(base + 3 * _G, _LANES)] = mv1
        return grp_body

    @pl.when(wid < nblk_total)
    def _():
        in_copy(wid, 0)

    tab_cp[0].wait()
    tab_cp[1].wait()

    for k in range(kmax):
        cur = k % 2
        b = k * _NW + wid

        if k + 1 < kmax:
            bn = (k + 1) * _NW + wid

            @pl.when(bn < nblk_total)
            def _(bn=bn, nxt=1 - cur):
                in_copy(bn, nxt)

        @pl.when(b < nblk_total)
        def _(k=k, b=b, cur=cur):
            pltpu.make_async_copy(uv_hbm.at[pl.ds(b * 2 * B, 2 * B)],
                                  uv_v.at[cur], sem_in[cur]).wait()
            if k >= 2:
                bp = (k - 2) * _NW + wid
                slp = pl.ds(bp * B * 4, B * 4)
                pltpu.make_async_copy(
                    o_v.at[cur], out_hbm.at[slp], sem_out[cur]).wait()
            plsc.parallel_loop(0, B // _G, unroll=7)(make_grp_body(cur))
            pltpu.async_copy(o_v.at[cur], out_hbm.at[pl.ds(b * B * 4, B * 4)],
                             sem_out[cur])

    for k in (kmax - 2, kmax - 1):
        if k >= 0:
            b = k * _NW + wid

            @pl.when(b < nblk_total)
            def _(k=k, b=b):
                pltpu.make_async_copy(
                    o_v.at[k % 2], out_hbm.at[pl.ds(b * B * 4, B * 4)],
                    sem_out[k % 2]).wait()


def kernel(uv, m_u, m_v):
    N = uv.shape[0]
    L = m_u.shape[1]
    G = _G
    B = 4480
    Np = ((N + B * _NW - 1) // (B * _NW)) * (B * _NW)
    nblk_total = Np // B

    pad = Np - N
    uvp = jnp.pad(uv, ((0, pad), (0, 0)))
    uvx = uvp.reshape(Np // G, G, 2).transpose(0, 2, 1).reshape(2 * Np)

    mesh = plsc.VectorSubcoreMesh(core_axis_name="c", subcore_axis_name="s")
    f = pl.kernel(
        functools.partial(_lpe_body, L, B, nblk_total),
        out_type=jax.ShapeDtypeStruct((Np * 4,), jnp.float32),
        mesh=mesh,
        compiler_params=pltpu.CompilerParams(
            needs_layout_passes=False, use_tc_tiling_on_sc=False),
        scratch_types=[
            pltpu.VMEM((2, 2 * B), jnp.float32),
            pltpu.VMEM((2, 4 * B), jnp.float32),
            pltpu.VMEM((2 * L,), jnp.float32),
            pltpu.VMEM((2 * L,), jnp.float32),
            (pltpu.SemaphoreType.DMA, pltpu.SemaphoreType.DMA),
            (pltpu.SemaphoreType.DMA, pltpu.SemaphoreType.DMA),
            pltpu.SemaphoreType.DMA,
        ],
    )
    out_flat = f(uvx, m_u.reshape(2 * L), m_v.reshape(2 * L))
    out = out_flat.reshape(Np // G, 4, G).transpose(0, 2, 1).reshape(Np, 4)
    return out[:N]
